# Initial kernel scaffold; baseline (speedup 1.0000x reference)
import jax
import jax.numpy as jnp
from jax.experimental import pallas as pl


def kernel(x, edge_index, enc_W0, enc_b0, gcn_W0, gcn_b0, gcn_W1, gcn_b1, dec_W0, dec_b0):
    raise NotImplementedError("write your pallas kernel here")


# speedup vs baseline: 9.4750x; 9.4750x over previous
import functools

import jax
import jax.numpy as jnp
from jax import lax
from jax.experimental import pallas as pl
from jax.experimental.pallas import tpu as pltpu
from jax.experimental.pallas import tpu_sc as plsc

N = 10000
E = 320000
D = 128
C = 40

NC, NS = 2, 16
NW = NC * NS
E_PER_W = E // NW
CHUNK = 80
NCH = E_PER_W // CHUNK
NP = 10112
RPS = NP // NS
PH = 64

BLK = 1000

_SC_MESH = plsc.VectorSubcoreMesh(core_axis_name="c", subcore_axis_name="s")


@functools.partial(
    pl.kernel,
    out_type=jax.ShapeDtypeStruct((NC * NP, D), jnp.float32),
    mesh=_SC_MESH,
    scratch_types=[
        pltpu.VMEM((PH, CHUNK), jnp.int32),
        pltpu.VMEM((PH, CHUNK), jnp.int32),
        pltpu.VMEM((CHUNK, D), jnp.float32),
        pltpu.VMEM((CHUNK, D), jnp.float32),
        pltpu.VMEM_SHARED((NP, D), jnp.float32),
        pltpu.SemaphoreType.DMA,
        pltpu.SemaphoreType.DMA,
    ],
)
def _sc_aggregate(m_hbm, src_hbm, dst_hbm, z_hbm, out_hbm,
                  sidx, didx, buf0, buf1, acc, sem0, sem1):
    cid = lax.axis_index("c")
    sid = lax.axis_index("s")
    wid = cid * NS + sid

    pltpu.sync_copy(z_hbm, acc.at[pl.ds(sid * RPS, RPS)])
    plsc.subcore_barrier()

    def run_phase(base, count):
        pltpu.sync_copy(src_hbm.at[wid, pl.ds(base, count)],
                        sidx.at[pl.ds(0, count)])
        pltpu.sync_copy(dst_hbm.at[wid, pl.ds(base, count)],
                        didx.at[pl.ds(0, count)])
        pltpu.async_copy(m_hbm.at[sidx.at[0]], buf0, sem0)

        @pl.loop(0, count - (count % 2), step=2)
        def _(j):
            cp1 = pltpu.async_copy(m_hbm.at[sidx.at[j + 1]], buf1, sem1)
            pltpu.make_async_copy(m_hbm.at[sidx.at[j]], buf0, sem0).wait()
            pltpu.sync_copy(buf0, acc.at[didx.at[j]], add=True)

            @pl.when(j + 2 < count)
            def _():
                pltpu.async_copy(m_hbm.at[sidx.at[j + 2]], buf0, sem0)

            cp1.wait()
            pltpu.sync_copy(buf1, acc.at[didx.at[j + 1]], add=True)

        if count % 2:
            pltpu.make_async_copy(m_hbm.at[sidx.at[count - 1]], buf0,
                                  sem0).wait()
            pltpu.sync_copy(buf0, acc.at[didx.at[count - 1]], add=True)

    run_phase(0, PH)
    run_phase(PH, NCH - PH)

    plsc.subcore_barrier()
    pltpu.sync_copy(acc.at[pl.ds(sid * RPS, RPS)],
                    out_hbm.at[pl.ds(cid * NP + sid * RPS, RPS)])


def _tc_encode(x, enc_W0, enc_b0, gcn_W0):
    def body(x_ref, w0_ref, b0_ref, w1_ref, o_ref):
        h = jnp.dot(x_ref[...], w0_ref[...],
                    preferred_element_type=jnp.float32) + b0_ref[...]
        o_ref[...] = jnp.dot(h, w1_ref[...], preferred_element_type=jnp.float32)

    return pl.pallas_call(
        body,
        grid=(N // BLK,),
        in_specs=[
            pl.BlockSpec((BLK, D), lambda i: (i, 0)),
            pl.BlockSpec((D, D), lambda i: (0, 0)),
            pl.BlockSpec((1, D), lambda i: (0, 0)),
            pl.BlockSpec((D, D), lambda i: (0, 0)),
        ],
        out_specs=pl.BlockSpec((BLK, D), lambda i: (i, 0)),
        out_shape=jax.ShapeDtypeStruct((N, D), jnp.float32),
    )(x, enc_W0, enc_b0, gcn_W0)


def _tc_conv_out(p0, p1, b, W):
    def body(p0_ref, p1_ref, b_ref, w_ref, o_ref):
        h = jax.nn.relu(p0_ref[...] + p1_ref[...] + b_ref[...])
        o_ref[...] = jnp.dot(h, w_ref[...], preferred_element_type=jnp.float32)

    return pl.pallas_call(
        body,
        grid=(N // BLK,),
        in_specs=[
            pl.BlockSpec((BLK, D), lambda i: (i, 0)),
            pl.BlockSpec((BLK, D), lambda i: (i, 0)),
            pl.BlockSpec((1, D), lambda i: (0, 0)),
            pl.BlockSpec((D, D), lambda i: (0, 0)),
        ],
        out_specs=pl.BlockSpec((BLK, D), lambda i: (i, 0)),
        out_shape=jax.ShapeDtypeStruct((N, D), jnp.float32),
    )(p0, p1, b, W)


def _tc_decode(q0, q1, b, decW_pad, dec_b_pad):
    def body(q0_ref, q1_ref, b_ref, w_ref, db_ref, o_ref):
        h = jax.nn.relu(q0_ref[...] + q1_ref[...] + b_ref[...])
        logits = jnp.dot(h, w_ref[...],
                         preferred_element_type=jnp.float32) + db_ref[...]
        mx = jnp.max(logits, axis=1, keepdims=True)
        lse = jnp.log(jnp.sum(jnp.exp(logits - mx), axis=1, keepdims=True))
        o_ref[...] = logits - mx - lse

    return pl.pallas_call(
        body,
        grid=(N // BLK,),
        in_specs=[
            pl.BlockSpec((BLK, D), lambda i: (i, 0)),
            pl.BlockSpec((BLK, D), lambda i: (i, 0)),
            pl.BlockSpec((1, D), lambda i: (0, 0)),
            pl.BlockSpec((D, D), lambda i: (0, 0)),
            pl.BlockSpec((1, D), lambda i: (0, 0)),
        ],
        out_specs=pl.BlockSpec((BLK, D), lambda i: (i, 0)),
        out_shape=jax.ShapeDtypeStruct((N, D), jnp.float32),
    )(q0, q1, b, decW_pad, dec_b_pad)


def kernel(x, edge_index, enc_W0, enc_b0, gcn_W0, gcn_b0, gcn_W1, gcn_b1,
           dec_W0, dec_b0):
    src = edge_index[0].astype(jnp.int32).reshape(NW, NCH, CHUNK)
    dst = edge_index[1].astype(jnp.int32).reshape(NW, NCH, CHUNK)
    zeros = jnp.zeros((RPS, D), jnp.float32)

    decW_pad = jnp.zeros((D, D), jnp.float32).at[:, :C].set(dec_W0)
    dec_b_pad = jnp.full((D,), -1e30, jnp.float32).at[:C].set(dec_b0)

    m0 = _tc_encode(x, enc_W0, enc_b0.reshape(1, D), gcn_W0)
    p = _sc_aggregate(m0, src, dst, zeros)
    m1 = _tc_conv_out(p[:N], p[NP:NP + N], gcn_b0.reshape(1, D), gcn_W1)
    q = _sc_aggregate(m1, src, dst, zeros)
    full = _tc_decode(q[:N], q[NP:NP + N], gcn_b1.reshape(1, D),
                      decW_pad, dec_b_pad.reshape(1, D))
    return full[:, :C]
